# Initial kernel scaffold; baseline (speedup 1.0000x reference)
import jax
import jax.numpy as jnp
from jax.experimental import pallas as pl


def kernel(x, edge_attr, atom_tables, atom_mixer, edge_tables, edge_mixer):
    raise NotImplementedError("write your pallas kernel here")


# speedup vs baseline: 4.8993x; 4.8993x over previous
import functools

import jax
import jax.numpy as jnp
import numpy as np
from jax.experimental import pallas as pl

_FEAT_DIMS = [119, 10, 11, 12, 9, 5, 8, 2, 2]
_EDGE_DIMS = [22, 6, 2]


def _fused_body(x_ref, tab_ref, w1_ref, b1_ref, g1_ref, bb1_ref,
                w2_ref, b2_ref, g2_ref, bb2_ref, o_ref,
                *, offsets, vocab_pad, block_rows):
    idx = x_ref[...]
    iota = jax.lax.broadcasted_iota(jnp.int32, (block_rows, vocab_pad), 1)
    oh = jnp.zeros((block_rows, vocab_pad), jnp.float32)
    for i, off in enumerate(offsets):
        oh = oh + (iota == idx[:, i][:, None] + off).astype(jnp.float32)
    emb = jnp.dot(oh, tab_ref[...], preferred_element_type=jnp.float32)

    h = jnp.dot(emb, w1_ref[...], preferred_element_type=jnp.float32)
    h = h + b1_ref[...]
    mu = jnp.mean(h, axis=-1, keepdims=True)
    var = jnp.mean((h - mu) ** 2, axis=-1, keepdims=True)
    h = (h - mu) * jax.lax.rsqrt(var + 1e-5) * g1_ref[...] + bb1_ref[...]
    h = jax.nn.gelu(h)

    out = jnp.dot(h, w2_ref[...], preferred_element_type=jnp.float32)
    out = out + b2_ref[...]
    mu = jnp.mean(out, axis=-1, keepdims=True)
    var = jnp.mean((out - mu) ** 2, axis=-1, keepdims=True)
    o_ref[...] = (out - mu) * jax.lax.rsqrt(var + 1e-5) * g2_ref[...] + bb2_ref[...]


def _embed_mix(idx, tables, mixer, dims, vocab_pad, block_rows):
    n_rows, n_feat = idx.shape
    d = tables[0].shape[1]
    tab = jnp.concatenate(tables, axis=0)
    tab = jnp.pad(tab, ((0, vocab_pad - tab.shape[0]), (0, 0)))
    offsets = tuple(int(v) for v in np.concatenate([[0], np.cumsum(dims[:-1])]))

    grid = (n_rows // block_rows,)
    row_spec = lambda shape: pl.BlockSpec(shape, lambda i: (i, 0))
    rep_spec = lambda shape: pl.BlockSpec(shape, lambda i: (0, 0))

    body = functools.partial(_fused_body, offsets=offsets,
                             vocab_pad=vocab_pad, block_rows=block_rows)
    return pl.pallas_call(
        body,
        grid=grid,
        in_specs=[
            row_spec((block_rows, n_feat)),
            rep_spec((vocab_pad, d)),
            rep_spec((d, 2 * d)),
            rep_spec((1, 2 * d)),
            rep_spec((1, 2 * d)),
            rep_spec((1, 2 * d)),
            rep_spec((2 * d, d)),
            rep_spec((1, d)),
            rep_spec((1, d)),
            rep_spec((1, d)),
        ],
        out_specs=row_spec((block_rows, d)),
        out_shape=jax.ShapeDtypeStruct((n_rows, d), jnp.float32),
    )(idx, tab,
      mixer['W1'], mixer['b1'][None, :], mixer['ln1_g'][None, :],
      mixer['ln1_b'][None, :],
      mixer['W2'], mixer['b2'][None, :], mixer['ln2_g'][None, :],
      mixer['ln2_b'][None, :])


def kernel(x, edge_attr, atom_tables, atom_mixer, edge_tables, edge_mixer):
    x_embedding = _embed_mix(x, atom_tables, atom_mixer, _FEAT_DIMS,
                             vocab_pad=256, block_rows=1000)
    edge_embedding = _embed_mix(edge_attr, edge_tables, edge_mixer, _EDGE_DIMS,
                                vocab_pad=32, block_rows=4000)
    return (x_embedding, edge_embedding)
